# Initial kernel scaffold; baseline (speedup 1.0000x reference)
import jax
import jax.numpy as jnp
from jax.experimental import pallas as pl


def kernel(x, src, tgt, Mtgt, W, b):
    raise NotImplementedError("write your pallas kernel here")


# speedup vs baseline: 3.8159x; 3.8159x over previous
import jax
import jax.numpy as jnp
from jax import lax
from jax.experimental import pallas as pl
from jax.experimental.pallas import tpu as pltpu
from jax.experimental.pallas import tpu_sc as plsc

_N = 10000
_NPAD = 10240
_E = 320000
_D = 128
_STEPS = 8
_DT = 1.0 / _STEPS

_NC = 2
_NS = 16
_NW = _NC * _NS
_EPW = _E // _NW
_C = 80
_NCH = _EPW // _C
_RPS = _NPAD // _NS
_ZR = 128

_LANES = 16
_DSUB = _D // _LANES


def _sc_body(y_hbm, src_hbm, tgt_hbm, m_hbm, out_hbm,
             src_c, tgt_c, m_c, rows_v, zero_v, agg_s, sem):
    cid = lax.axis_index("c")
    sid = lax.axis_index("s")
    wid = cid * _NS + sid

    @pl.loop(0, _ZR)
    def _zb(i):
        z = jnp.zeros((_LANES,), jnp.float32)
        for j in range(_DSUB):
            zero_v[i, pl.ds(j * _LANES, _LANES)] = z
    for r in range(_RPS // _ZR):
        pltpu.sync_copy(zero_v, agg_s.at[pl.ds(sid * _RPS + r * _ZR, _ZR)])
    plsc.subcore_barrier()

    @pl.loop(0, _NCH)
    def _chunk(ci):
        base = wid * _EPW + ci * _C
        pltpu.sync_copy(src_hbm.at[pl.ds(base, _C)], src_c)
        pltpu.sync_copy(tgt_hbm.at[pl.ds(base, _C)], tgt_c)
        pltpu.sync_copy(m_hbm.at[pl.ds(base, _C)], m_c)
        pltpu.async_copy(y_hbm.at[src_c], rows_v, sem).wait()

        @pl.loop(0, _C // _LANES)
        def _scale(g):
            mvec = m_c[pl.ds(g * _LANES, _LANES)]
            for e in range(_LANES):
                m = mvec[e]
                row = g * _LANES + e
                for j in range(_DSUB):
                    sl = pl.ds(j * _LANES, _LANES)
                    rows_v[row, sl] = rows_v[row, sl] * m

        pltpu.sync_copy(rows_v, agg_s.at[tgt_c], add=True)

    plsc.subcore_barrier()
    pltpu.sync_copy(agg_s.at[pl.ds(sid * _RPS, _RPS)],
                    out_hbm.at[cid, pl.ds(sid * _RPS, _RPS)])


_sc_agg = pl.kernel(
    _sc_body,
    out_type=jax.ShapeDtypeStruct((_NC, _NPAD, _D), jnp.float32),
    mesh=plsc.VectorSubcoreMesh(core_axis_name="c", subcore_axis_name="s"),
    scratch_types=[
        pltpu.VMEM((_C,), jnp.int32),
        pltpu.VMEM((_C,), jnp.int32),
        pltpu.VMEM((_C,), jnp.float32),
        pltpu.VMEM((_C, _D), jnp.float32),
        pltpu.VMEM((_ZR, _D), jnp.float32),
        pltpu.VMEM_SHARED((_NPAD, _D), jnp.float32),
        pltpu.SemaphoreType.DMA,
    ],
)

_sc_agg = jax.jit(_sc_agg)


_BN = 1024


def _stage_c(s_ref, w_ref, b_ref):
    return jnp.tanh(
        jnp.dot(s_ref[0] + s_ref[1], w_ref[...],
                preferred_element_type=jnp.float32) + b_ref[...])


def _tc_first_body(s_ref, h_ref, w_ref, b_ref, y_ref, acc_ref):
    c = _stage_c(s_ref, w_ref, b_ref)
    y_ref[...] = h_ref[...] + (_DT / 2.0) * c
    acc_ref[...] = c


def _mk_mid_body(alpha):
    def body(s_ref, h_ref, a_ref, w_ref, b_ref, y_ref, ao_ref):
        c = _stage_c(s_ref, w_ref, b_ref)
        y_ref[...] = h_ref[...] + alpha * c
        ao_ref[...] = a_ref[...] + 2.0 * c
    return body


def _tc_last_body(s_ref, h_ref, a_ref, w_ref, b_ref, y_ref):
    c = _stage_c(s_ref, w_ref, b_ref)
    y_ref[...] = h_ref[...] + (_DT / 6.0) * (a_ref[...] + c)


_row_spec = pl.BlockSpec((_BN, _D), lambda i: (i, 0))
_s_spec = pl.BlockSpec((_NC, _BN, _D), lambda i: (0, i, 0))
_w_spec = pl.BlockSpec((_D, _D), lambda i: (0, 0))
_b_spec = pl.BlockSpec((1, _D), lambda i: (0, 0))
_out_nd = jax.ShapeDtypeStruct((_NPAD, _D), jnp.float32)

_tc_first = pl.pallas_call(
    _tc_first_body,
    grid=(_NPAD // _BN,),
    in_specs=[_s_spec, _row_spec, _w_spec, _b_spec],
    out_specs=[_row_spec, _row_spec],
    out_shape=[_out_nd, _out_nd],
)

_tc_mid_half = pl.pallas_call(
    _mk_mid_body(_DT / 2.0),
    grid=(_NPAD // _BN,),
    in_specs=[_s_spec, _row_spec, _row_spec, _w_spec, _b_spec],
    out_specs=[_row_spec, _row_spec],
    out_shape=[_out_nd, _out_nd],
)

_tc_mid_full = pl.pallas_call(
    _mk_mid_body(_DT),
    grid=(_NPAD // _BN,),
    in_specs=[_s_spec, _row_spec, _row_spec, _w_spec, _b_spec],
    out_specs=[_row_spec, _row_spec],
    out_shape=[_out_nd, _out_nd],
)

_tc_last = pl.pallas_call(
    _tc_last_body,
    grid=(_NPAD // _BN,),
    in_specs=[_s_spec, _row_spec, _row_spec, _w_spec, _b_spec],
    out_specs=[_row_spec],
    out_shape=[_out_nd],
)


def kernel(x, src, tgt, Mtgt, W, b):
    b2 = b.reshape(1, _D)

    h0 = jnp.concatenate(
        [x, jnp.zeros((_NPAD - _N, _D), jnp.float32)], axis=0)

    def _step(h, _):
        s = _sc_agg(h, src, tgt, Mtgt)
        y, acc = _tc_first(s, h, W, b2)
        s = _sc_agg(y, src, tgt, Mtgt)
        y, acc = _tc_mid_half(s, h, acc, W, b2)
        s = _sc_agg(y, src, tgt, Mtgt)
        y, acc = _tc_mid_full(s, h, acc, W, b2)
        s = _sc_agg(y, src, tgt, Mtgt)
        return _tc_last(s, h, acc, W, b2)[0], None

    h, _ = lax.scan(_step, h0, None, length=_STEPS)
    return h[:_N]


# speedup vs baseline: 9.4937x; 2.4880x over previous
import jax
import jax.numpy as jnp
from jax import lax
from jax.experimental import pallas as pl
from jax.experimental.pallas import tpu as pltpu
from jax.experimental.pallas import tpu_sc as plsc

_N = 10000
_NPAD = 10240
_E = 320000
_D = 128
_STEPS = 8
_DT = 1.0 / _STEPS

_NC = 2
_NS = 16
_NW = _NC * _NS
_EPW = _E // _NW
_C = 80
_NCH = _EPW // _C
_RPS = _NPAD // _NS
_ZR = 128

_LANES = 16
_DSUB = _D // _LANES


_NB = 2


def _sc_body(y_hbm, src_hbm, tgt_hbm, m_hbm, out_hbm,
             sg0, sg1, tg0, tg1, mg0, mg1,
             rg0, rg1, rs0, rs1,
             gsem0, gsem1, stsem0, stsem1, ttsem0, ttsem1, ssem0, ssem1,
             agg_s):
    sg = (sg0, sg1)
    tg = (tg0, tg1)
    mg = (mg0, mg1)
    rg = (rg0, rg1)
    rs = (rs0, rs1)
    gsem = (gsem0, gsem1)
    stsem = (stsem0, stsem1)
    ttsem = (ttsem0, ttsem1)
    ssem = (ssem0, ssem1)

    cid = lax.axis_index("c")
    sid = lax.axis_index("s")
    wid = cid * _NS + sid
    ebase = wid * _EPW

    @pl.loop(0, _C)
    def _zb(i):
        z = jnp.zeros((_LANES,), jnp.float32)
        for j in range(_DSUB):
            rs0[i, pl.ds(j * _LANES, _LANES)] = z
    for r in range(_RPS // _C):
        pltpu.sync_copy(rs0, agg_s.at[pl.ds(sid * _RPS + r * _C, _C)])
    plsc.subcore_barrier()

    def _stage_sm(c, k):
        pltpu.async_copy(src_hbm.at[pl.ds(ebase + c * _C, _C)],
                         sg[k], stsem[k])
        pltpu.async_copy(m_hbm.at[pl.ds(ebase + c * _C, _C)],
                         mg[k], stsem[k])

    def _wait_stage_sm(c, k):
        pltpu.make_async_copy(src_hbm.at[pl.ds(ebase + c * _C, _C)],
                              sg[k], stsem[k]).wait()
        pltpu.make_async_copy(m_hbm.at[pl.ds(ebase + c * _C, _C)],
                              mg[k], stsem[k]).wait()

    def _process(c, k, in_loop):
        pltpu.make_async_copy(y_hbm.at[sg[k]], rg[k], gsem[k]).wait()

        if in_loop:
            @pl.when(c + 1 <= _NCH - 1)
            def _():
                _wait_stage_sm(c + 1, 1 - k)
                pltpu.async_copy(y_hbm.at[sg[1 - k]], rg[1 - k],
                                 gsem[1 - k])

        @pl.when(c >= 2)
        def _():
            pltpu.make_async_copy(rs[k], agg_s.at[tg[k]], ssem[k]).wait()
        pltpu.async_copy(tgt_hbm.at[pl.ds(ebase + c * _C, _C)],
                         tg[k], ttsem[k])

        @plsc.parallel_loop(0, _C // _LANES)
        def _scale(g):
            mvec = mg[k][pl.ds(g * _LANES, _LANES)]
            for e in range(_LANES):
                m = mvec[e]
                row = g * _LANES + e
                for j in range(_DSUB):
                    sl = pl.ds(j * _LANES, _LANES)
                    rs[k][row, sl] = rg[k][row, sl] * m

        pltpu.make_async_copy(tgt_hbm.at[pl.ds(ebase + c * _C, _C)],
                              tg[k], ttsem[k]).wait()
        pltpu.async_copy(rs[k], agg_s.at[tg[k]], ssem[k], add=True)

        if in_loop:
            @pl.when(c + 2 <= _NCH - 1)
            def _():
                _stage_sm(c + 2, k)

    _stage_sm(0, 0)
    _stage_sm(1, 1)
    _wait_stage_sm(0, 0)
    pltpu.async_copy(y_hbm.at[sg0], rg0, gsem0)

    @pl.loop(0, _NCH - 1, step=_NB)
    def _outer(ci):
        for k in range(_NB):
            _process(ci + k, k, True)

    _process(_NCH - 1, 0, False)

    pltpu.make_async_copy(rs1, agg_s.at[tg1], ssem1).wait()
    pltpu.make_async_copy(rs0, agg_s.at[tg0], ssem0).wait()

    plsc.subcore_barrier()
    pltpu.sync_copy(agg_s.at[pl.ds(sid * _RPS, _RPS)],
                    out_hbm.at[cid, pl.ds(sid * _RPS, _RPS)])


_sc_agg = pl.kernel(
    _sc_body,
    out_type=jax.ShapeDtypeStruct((_NC, _NPAD, _D), jnp.float32),
    mesh=plsc.VectorSubcoreMesh(core_axis_name="c", subcore_axis_name="s"),
    scratch_types=(
        [pltpu.VMEM((_C,), jnp.int32) for _ in range(_NB)]
        + [pltpu.VMEM((_C,), jnp.int32) for _ in range(_NB)]
        + [pltpu.VMEM((_C,), jnp.float32) for _ in range(_NB)]
        + [pltpu.VMEM((_C, _D), jnp.float32) for _ in range(_NB)]
        + [pltpu.VMEM((_C, _D), jnp.float32) for _ in range(_NB)]
        + [pltpu.SemaphoreType.DMA for _ in range(4 * _NB)]
        + [pltpu.VMEM_SHARED((_NPAD, _D), jnp.float32)]
    ),
)

_sc_agg = jax.jit(_sc_agg)


_BN = 1024


def _stage_c(s_ref, w_ref, b_ref):
    return jnp.tanh(
        jnp.dot(s_ref[0] + s_ref[1], w_ref[...],
                preferred_element_type=jnp.float32) + b_ref[...])


def _tc_first_body(s_ref, h_ref, w_ref, b_ref, y_ref, acc_ref):
    c = _stage_c(s_ref, w_ref, b_ref)
    y_ref[...] = h_ref[...] + (_DT / 2.0) * c
    acc_ref[...] = c


def _mk_mid_body(alpha):
    def body(s_ref, h_ref, a_ref, w_ref, b_ref, y_ref, ao_ref):
        c = _stage_c(s_ref, w_ref, b_ref)
        y_ref[...] = h_ref[...] + alpha * c
        ao_ref[...] = a_ref[...] + 2.0 * c
    return body


def _tc_last_body(s_ref, h_ref, a_ref, w_ref, b_ref, y_ref):
    c = _stage_c(s_ref, w_ref, b_ref)
    y_ref[...] = h_ref[...] + (_DT / 6.0) * (a_ref[...] + c)


_row_spec = pl.BlockSpec((_BN, _D), lambda i: (i, 0))
_s_spec = pl.BlockSpec((_NC, _BN, _D), lambda i: (0, i, 0))
_w_spec = pl.BlockSpec((_D, _D), lambda i: (0, 0))
_b_spec = pl.BlockSpec((1, _D), lambda i: (0, 0))
_out_nd = jax.ShapeDtypeStruct((_NPAD, _D), jnp.float32)

_tc_first = pl.pallas_call(
    _tc_first_body,
    grid=(_NPAD // _BN,),
    in_specs=[_s_spec, _row_spec, _w_spec, _b_spec],
    out_specs=[_row_spec, _row_spec],
    out_shape=[_out_nd, _out_nd],
)

_tc_mid_half = pl.pallas_call(
    _mk_mid_body(_DT / 2.0),
    grid=(_NPAD // _BN,),
    in_specs=[_s_spec, _row_spec, _row_spec, _w_spec, _b_spec],
    out_specs=[_row_spec, _row_spec],
    out_shape=[_out_nd, _out_nd],
)

_tc_mid_full = pl.pallas_call(
    _mk_mid_body(_DT),
    grid=(_NPAD // _BN,),
    in_specs=[_s_spec, _row_spec, _row_spec, _w_spec, _b_spec],
    out_specs=[_row_spec, _row_spec],
    out_shape=[_out_nd, _out_nd],
)

_tc_last = pl.pallas_call(
    _tc_last_body,
    grid=(_NPAD // _BN,),
    in_specs=[_s_spec, _row_spec, _row_spec, _w_spec, _b_spec],
    out_specs=[_row_spec],
    out_shape=[_out_nd],
)


def kernel(x, src, tgt, Mtgt, W, b):
    b2 = b.reshape(1, _D)

    h0 = jnp.concatenate(
        [x, jnp.zeros((_NPAD - _N, _D), jnp.float32)], axis=0)

    def _step(h, _):
        s = _sc_agg(h, src, tgt, Mtgt)
        y, acc = _tc_first(s, h, W, b2)
        s = _sc_agg(y, src, tgt, Mtgt)
        y, acc = _tc_mid_half(s, h, acc, W, b2)
        s = _sc_agg(y, src, tgt, Mtgt)
        y, acc = _tc_mid_full(s, h, acc, W, b2)
        s = _sc_agg(y, src, tgt, Mtgt)
        return _tc_last(s, h, acc, W, b2)[0], None

    h, _ = lax.scan(_step, h0, None, length=_STEPS)
    return h[:_N]
